# baseline (speedup 1.0000x reference)
import math

import jax
import jax.numpy as jnp
from jax import lax
from jax.experimental import pallas as pl
from jax.experimental.pallas import tpu as pltpu
from jax.experimental.pallas import tpu_sc as plsc

VOCAB = 1000000
D = 64
SCALE = math.sqrt(D)

NC, NS = 2, 16
NW = NC * NS
BLK = 128
KB = 4
CHUNK = KB * BLK


def _make_kernel(n_total):
    per_w = n_total // NW
    n_chunks = per_w // CHUNK
    mesh = plsc.VectorSubcoreMesh(core_axis_name="c", subcore_axis_name="s")

    def body(table_hbm, idx_hbm, out_hbm, idx_v, rows_v, gsem):
        c = lax.axis_index("c")
        s = lax.axis_index("s")
        wid = s * NC + c
        base = wid * per_w

        def chunk(g, carry):
            pltpu.sync_copy(idx_hbm.at[wid, g], idx_v)
            copies = [
                pltpu.async_copy(
                    table_hbm.at[idx_v.at[j]],
                    rows_v.at[pl.ds(j * BLK, BLK)],
                    gsem,
                )
                for j in range(KB)
            ]
            for cp in copies:
                cp.wait()

            def srow(r, carry2):
                for j in range(D // 16):
                    rows_v[r, pl.ds(j * 16, 16)] = (
                        rows_v[r, pl.ds(j * 16, 16)] * SCALE
                    )
                return carry2

            lax.fori_loop(0, CHUNK, srow, 0)
            pltpu.sync_copy(rows_v, out_hbm.at[pl.ds(base + g * CHUNK, CHUNK)])
            return carry

        lax.fori_loop(0, n_chunks, chunk, 0)

    return pl.kernel(
        body,
        out_type=jax.ShapeDtypeStruct((n_total, D), jnp.float32),
        mesh=mesh,
        scratch_types=[
            pltpu.VMEM((KB, BLK), jnp.int32),
            pltpu.VMEM((CHUNK, D), jnp.float32),
            pltpu.SemaphoreType.DMA,
        ],
        compiler_params=pltpu.CompilerParams(use_tc_tiling_on_sc=False),
    )


def kernel(x, table):
    b, l = x.shape
    n = b * l
    idx = x.reshape(NW, n // (NW * CHUNK), KB, BLK).astype(jnp.int32)
    out = _make_kernel(n)(table, idx)
    return out.reshape(b, l, D)


# speedup vs baseline: 1.1352x; 1.1352x over previous
import math

import jax
import jax.numpy as jnp
from jax import lax
from jax.experimental import pallas as pl
from jax.experimental.pallas import tpu as pltpu
from jax.experimental.pallas import tpu_sc as plsc

VOCAB = 1000000
D = 64
SCALE = math.sqrt(D)

NC, NS = 2, 16
NW = NC * NS
BLK = 128
KB = 4
CHUNK = KB * BLK
CHUNK_BYTES = CHUNK * D * 4


def _make_kernel(n_total):
    per_w = n_total // NW
    n_chunks = per_w // CHUNK
    n_blocks = per_w // BLK
    mesh = plsc.VectorSubcoreMesh(core_axis_name="c", subcore_axis_name="s")

    def body(table_hbm, idx_hbm, out_hbm, idx_all, rows0, rows1, g0, g1, o0, o1):
        c = lax.axis_index("c")
        s = lax.axis_index("s")
        wid = s * NC + c
        base = wid * per_w
        rows = [rows0, rows1]
        gsem = [g0, g1]
        osem = [o0, o1]

        pltpu.sync_copy(idx_hbm.at[wid], idx_all)

        def fire(g, b):
            for j in range(KB):
                pltpu.async_copy(
                    table_hbm.at[idx_all.at[g * KB + j]],
                    rows[b].at[pl.ds(j * BLK, BLK)],
                    gsem[b],
                )

        def drain(sem, b):
            pltpu.make_async_copy(
                table_hbm.at[pl.ds(0, CHUNK)], rows[b], sem[b]
            ).wait()

        def scale(b):
            rb = rows[b]

            @plsc.parallel_loop(0, CHUNK, step=1, unroll=8)
            def _(r):
                for j in range(D // 16):
                    rb[r, pl.ds(j * 16, 16)] = rb[r, pl.ds(j * 16, 16)] * SCALE

        def consume(g, b):
            drain(gsem, b)
            scale(b)
            pltpu.async_copy(
                rows[b], out_hbm.at[pl.ds(base + g * CHUNK, CHUNK)], osem[b]
            )

        fire(0, 0)

        def pair(gg, carry):
            for b in range(2):
                g = gg * 2 + b
                other = 1 - b

                @pl.when(g + 1 < n_chunks)
                def _():
                    @pl.when(g >= 1)
                    def _():
                        drain(osem, other)

                    fire(g + 1, other)

                consume(g, b)
            return carry

        lax.fori_loop(0, n_chunks // 2, pair, 0)
        drain(osem, 0)
        drain(osem, 1)

    return pl.kernel(
        body,
        out_type=jax.ShapeDtypeStruct((n_total, D), jnp.float32),
        mesh=mesh,
        scratch_types=[
            pltpu.VMEM((n_blocks, BLK), jnp.int32),
            pltpu.VMEM((CHUNK, D), jnp.float32),
            pltpu.VMEM((CHUNK, D), jnp.float32),
            pltpu.SemaphoreType.DMA,
            pltpu.SemaphoreType.DMA,
            pltpu.SemaphoreType.DMA,
            pltpu.SemaphoreType.DMA,
        ],
        compiler_params=pltpu.CompilerParams(use_tc_tiling_on_sc=False),
    )


def kernel(x, table):
    b, l = x.shape
    n = b * l
    idx = x.reshape(NW, n // (NW * BLK), BLK).astype(jnp.int32)
    out = _make_kernel(n)(table, idx)
    return out.reshape(b, l, D)


# speedup vs baseline: 1.3866x; 1.2214x over previous
import math

import jax
import jax.numpy as jnp
from jax import lax
from jax.experimental import pallas as pl
from jax.experimental.pallas import tpu as pltpu
from jax.experimental.pallas import tpu_sc as plsc

D = 64
DP = 128
SCALE = math.sqrt(D)

NC, NS = 2, 16
NW = NC * NS
BLK = 128
KB = 2
CHUNK = KB * BLK
CHUNK_BYTES = CHUNK * DP * 4


def _make_kernel(n_total):
    per_w = n_total // NW
    n_chunks = per_w // CHUNK
    n_idx_rows = per_w // BLK
    mesh = plsc.VectorSubcoreMesh(core_axis_name="c", subcore_axis_name="s")

    def body(table_hbm, idx_hbm, out_hbm, idx_all, rows0, rows1, g0, g1, o0, o1):
        c = lax.axis_index("c")
        s = lax.axis_index("s")
        wid = s * NC + c
        base = wid * per_w
        rows = [rows0, rows1]
        gsem = [g0, g1]
        osem = [o0, o1]

        pltpu.sync_copy(idx_hbm.at[pl.ds(wid * n_idx_rows, n_idx_rows)], idx_all)

        def fire(g, b):
            for j in range(KB):
                pltpu.async_copy(
                    table_hbm.at[idx_all.at[g * KB + j]],
                    rows[b].at[pl.ds(j * BLK, BLK)],
                    gsem[b],
                )

        def drain(sem, b):
            pltpu.make_async_copy(
                table_hbm.at[pl.ds(0, CHUNK)], rows[b], sem[b]
            ).wait()

        def scale(b):
            rb = rows[b]

            @plsc.parallel_loop(0, CHUNK, step=1, unroll=8)
            def _(r):
                for j in range(D // 16):
                    rb[r, pl.ds(j * 16, 16)] = rb[r, pl.ds(j * 16, 16)] * SCALE

        def consume(g, b):
            drain(gsem, b)
            scale(b)
            pltpu.async_copy(
                rows[b], out_hbm.at[pl.ds(base + g * CHUNK, CHUNK)], osem[b]
            )

        fire(0, 0)

        def pair(gg, carry):
            for b in range(2):
                g = gg * 2 + b
                other = 1 - b

                @pl.when(g + 1 < n_chunks)
                def _():
                    @pl.when(g >= 1)
                    def _():
                        drain(osem, other)

                    fire(g + 1, other)

                consume(g, b)
            return carry

        lax.fori_loop(0, n_chunks // 2, pair, 0)
        drain(osem, 0)
        drain(osem, 1)

    return pl.kernel(
        body,
        out_type=jax.ShapeDtypeStruct((n_total, DP), jnp.float32),
        mesh=mesh,
        scratch_types=[
            pltpu.VMEM((n_idx_rows, BLK), jnp.int32),
            pltpu.VMEM((CHUNK, DP), jnp.float32),
            pltpu.VMEM((CHUNK, DP), jnp.float32),
            pltpu.SemaphoreType.DMA,
            pltpu.SemaphoreType.DMA,
            pltpu.SemaphoreType.DMA,
            pltpu.SemaphoreType.DMA,
        ],
        compiler_params=pltpu.CompilerParams(use_tc_tiling_on_sc=True),
    )


def kernel(x, table):
    b, l = x.shape
    n = b * l
    tpad = jnp.pad(table, ((0, 0), (0, DP - D)))
    idx = x.reshape(n // BLK, BLK)
    out = _make_kernel(n)(tpad, idx)
    return out[:, :D].reshape(b, l, D)


# speedup vs baseline: 1.3919x; 1.0039x over previous
import math

import jax
import jax.numpy as jnp
from jax import lax
from jax.experimental import pallas as pl
from jax.experimental.pallas import tpu as pltpu
from jax.experimental.pallas import tpu_sc as plsc

D = 64
DP = 128
SCALE = math.sqrt(D)

NC, NS = 2, 16
NW = NC * NS
BLK = 128
CHUNK = BLK
NBUF = 4


def _make_kernel(n_total):
    per_w = n_total // NW
    n_chunks = per_w // CHUNK
    n_idx_rows = per_w // BLK
    mesh = plsc.VectorSubcoreMesh(core_axis_name="c", subcore_axis_name="s")

    def body(table_hbm, idx_hbm, out_hbm, idx_all, *bufs_and_sems):
        rows = list(bufs_and_sems[:NBUF])
        gsem = list(bufs_and_sems[NBUF:2 * NBUF])
        osem = list(bufs_and_sems[2 * NBUF:3 * NBUF])
        c = lax.axis_index("c")
        s = lax.axis_index("s")
        wid = s * NC + c
        base = wid * per_w

        pltpu.sync_copy(idx_hbm.at[pl.ds(wid * n_idx_rows, n_idx_rows)], idx_all)

        def fire(g, b):
            pltpu.async_copy(
                table_hbm.at[idx_all.at[g]], rows[b], gsem[b]
            )

        def drain_gather(b):
            pltpu.make_async_copy(
                table_hbm.at[pl.ds(0, CHUNK)], rows[b], gsem[b]
            ).wait()

        def drain_out(b):
            pltpu.make_async_copy(
                table_hbm.at[pl.ds(0, CHUNK)], rows[b], osem[b]
            ).wait()

        def scale(b):
            rb = rows[b]

            @plsc.parallel_loop(0, CHUNK, step=1, unroll=8)
            def _(r):
                for j in range(D // 16):
                    rb[r, pl.ds(j * 16, 16)] = rb[r, pl.ds(j * 16, 16)] * SCALE

        def consume(g, b):
            drain_gather(b)
            scale(b)
            pltpu.async_copy(
                rows[b], out_hbm.at[pl.ds(base + g * CHUNK, CHUNK)], osem[b]
            )

        AHEAD = NBUF - 2
        for g0 in range(AHEAD):
            fire(g0, g0)

        def step(it, carry):
            for b in range(NBUF):
                g = it * NBUF + b
                bn = (b + AHEAD) % NBUF

                @pl.when(g + AHEAD < n_chunks)
                def _():
                    @pl.when(g >= NBUF - AHEAD)
                    def _():
                        drain_out(bn)

                    fire(g + AHEAD, bn)

                consume(g, b)
            return carry

        lax.fori_loop(0, n_chunks // NBUF, step, 0)
        for b in range(NBUF):
            drain_out(b)

    return pl.kernel(
        body,
        out_type=jax.ShapeDtypeStruct((n_total, DP), jnp.float32),
        mesh=mesh,
        scratch_types=(
            [pltpu.VMEM((n_idx_rows, BLK), jnp.int32)]
            + [pltpu.VMEM((CHUNK, DP), jnp.float32)] * NBUF
            + [pltpu.SemaphoreType.DMA] * (2 * NBUF)
        ),
        compiler_params=pltpu.CompilerParams(use_tc_tiling_on_sc=True),
    )


def kernel(x, table):
    b, l = x.shape
    n = b * l
    tpad = jnp.pad(table, ((0, 0), (0, DP - D)))
    idx = x.reshape(n // BLK, BLK)
    out = _make_kernel(n)(tpad, idx)
    return out[:, :D].reshape(b, l, D)
